# baseline (speedup 1.0000x reference)
import functools
import math

import jax
import jax.numpy as jnp
from jax import lax
from jax.experimental import pallas as pl
from jax.experimental.pallas import tpu as pltpu
from jax.experimental.pallas import tpu_sc as plsc

DIM = 1024
MAX_SEQ_LEN = 2048
PAD = 0
L = 16


def _pos_embedding(max_seq_len, dim):
    half_dim = dim // 2
    emb = math.log(10000.0) / (half_dim - 1)
    emb = jnp.exp(jnp.arange(half_dim, dtype=jnp.float32) * -emb)
    pos = jnp.arange(max_seq_len, dtype=jnp.float32)
    emb = pos[:, None] * emb[None, :]
    pe = jnp.concatenate([jnp.sin(emb), jnp.cos(emb)], axis=1)
    if dim % 2 == 1:
        pe = jnp.concatenate(
            [pe, jnp.zeros((max_seq_len, 1), dtype=jnp.float32)], axis=1)
    return pe


@functools.lru_cache(maxsize=None)
def _make_sc_kernel(BT, D, sl, R):
    info = plsc.get_sparse_core_info()
    NC, NS = info.num_cores, info.num_subcores
    NW = NC * NS
    assert BT % (NW * R) == 0
    rows_per_worker = BT // NW
    n_chunks = rows_per_worker // R
    scale = math.sqrt(D)
    mesh = plsc.VectorSubcoreMesh(core_axis_name="c", subcore_axis_name="s")

    @functools.partial(
        pl.kernel,
        mesh=mesh,
        out_type=jax.ShapeDtypeStruct((BT, D), jnp.float32),
        scratch_types=[
            pltpu.VMEM((R,), jnp.int32),
            pltpu.VMEM((R,), jnp.int32),
            pltpu.VMEM((R, D), jnp.float32),
            pltpu.VMEM((R, D), jnp.float32),
            pltpu.SemaphoreType.DMA,
            pltpu.SemaphoreType.DMA,
        ],
    )
    def k(idx_hbm, table_hbm, pe_hbm, out_hbm,
          idx_v, pidx_v, rows_v, pos_v, sem_t, sem_p):
        wid = lax.axis_index("s") * NC + lax.axis_index("c")
        base = wid * rows_per_worker

        def chunk_body(c, carry):
            flat = base + c * R
            pltpu.sync_copy(idx_hbm.at[pl.ds(flat, R)], idx_v)
            spos = lax.rem(flat, sl)
            for i in range(R // L):
                v = idx_v[pl.ds(i * L, L)]
                p = spos + i * L + lax.iota(jnp.int32, L)
                pidx_v[pl.ds(i * L, L)] = jnp.where(v == PAD, sl, p)
            cp_t = pltpu.async_copy(table_hbm.at[idx_v], rows_v, sem_t)
            cp_p = pltpu.async_copy(pe_hbm.at[pidx_v], pos_v, sem_p)
            cp_t.wait()
            cp_p.wait()

            def row_body(r, rcarry):
                for cv in range(D // L):
                    s_ = pl.ds(cv * L, L)
                    rows_v[r, s_] = rows_v[r, s_] * scale + pos_v[r, s_]
                return rcarry

            lax.fori_loop(0, R, row_body, 0)
            pltpu.sync_copy(rows_v, out_hbm.at[pl.ds(flat, R)])
            return carry

        lax.fori_loop(0, n_chunks, chunk_body, 0)

    return k


def kernel(input, embeddings, step=0):
    bs, sl = input.shape
    d = embeddings.shape[1]
    BT = bs * sl
    idx_flat = input.reshape(BT).astype(jnp.int32)
    pe = _pos_embedding(MAX_SEQ_LEN, d)
    pe_sl = lax.dynamic_slice_in_dim(pe, step, sl, axis=0)
    pe_aug = jnp.concatenate([pe_sl, jnp.zeros((8, d), jnp.float32)], axis=0)
    out_flat = _make_sc_kernel(BT, d, sl, 32)(idx_flat, embeddings, pe_aug)
    return out_flat.reshape(bs, sl, d)


# speedup vs baseline: 1.1790x; 1.1790x over previous
import functools
import math

import jax
import jax.numpy as jnp
from jax import lax
from jax.experimental import pallas as pl
from jax.experimental.pallas import tpu as pltpu
from jax.experimental.pallas import tpu_sc as plsc

DIM = 1024
MAX_SEQ_LEN = 2048
PAD = 0
L = 16
R = 16
NBUF = 2


def _pos_embedding(max_seq_len, dim):
    half_dim = dim // 2
    emb = math.log(10000.0) / (half_dim - 1)
    emb = jnp.exp(jnp.arange(half_dim, dtype=jnp.float32) * -emb)
    pos = jnp.arange(max_seq_len, dtype=jnp.float32)
    emb = pos[:, None] * emb[None, :]
    pe = jnp.concatenate([jnp.sin(emb), jnp.cos(emb)], axis=1)
    if dim % 2 == 1:
        pe = jnp.concatenate(
            [pe, jnp.zeros((max_seq_len, 1), dtype=jnp.float32)], axis=1)
    return pe


@functools.lru_cache(maxsize=None)
def _make_sc_kernel(BT, D, sl):
    info = plsc.get_sparse_core_info()
    NC, NS = info.num_cores, info.num_subcores
    NW = NC * NS
    assert BT % (NW * R) == 0
    rpw = BT // NW
    n_chunks = rpw // R
    assert n_chunks % NBUF == 0 and n_chunks >= 2 * NBUF
    n_groups = n_chunks // NBUF
    assert sl % rpw == 0
    scale = math.sqrt(D)
    mesh = plsc.VectorSubcoreMesh(core_axis_name="c", subcore_axis_name="s")

    @functools.partial(
        pl.kernel,
        mesh=mesh,
        out_type=jax.ShapeDtypeStruct((BT, D), jnp.float32),
        scratch_types=[
            pltpu.VMEM((rpw,), jnp.int32),
            pltpu.VMEM((rpw,), jnp.int32),
            pltpu.VMEM((NBUF, R, D), jnp.float32),
            pltpu.VMEM((NBUF, R, D), jnp.float32),
            pltpu.VMEM((NBUF, R, D), jnp.float32),
            pltpu.SemaphoreType.DMA((NBUF,)),
            pltpu.SemaphoreType.DMA((NBUF,)),
            pltpu.SemaphoreType.DMA((NBUF,)),
        ],
    )
    def k(idx_hbm, table_hbm, pe_hbm, out_hbm,
          idx_all, pidx_all, rows, pos, outb, sem_t, sem_p, sem_o):
        wid = lax.axis_index("s") * NC + lax.axis_index("c")
        base = wid * rpw
        spos0 = lax.rem(base, sl)

        pltpu.sync_copy(idx_hbm.at[pl.ds(base, rpw)], idx_all)
        for i in range(rpw // L):
            v = idx_all[pl.ds(i * L, L)]
            p = spos0 + i * L + lax.iota(jnp.int32, L)
            pidx_all[pl.ds(i * L, L)] = jnp.where(v == PAD, sl, p)

        def fire_gather(c, b):
            pltpu.async_copy(table_hbm.at[idx_all.at[pl.ds(c * R, R)]],
                             rows.at[b], sem_t.at[b])
            pltpu.async_copy(pe_hbm.at[pidx_all.at[pl.ds(c * R, R)]],
                             pos.at[b], sem_p.at[b])

        def wait_gather(c, b):
            pltpu.make_async_copy(table_hbm.at[idx_all.at[pl.ds(c * R, R)]],
                                  rows.at[b], sem_t.at[b]).wait()
            pltpu.make_async_copy(pe_hbm.at[pidx_all.at[pl.ds(c * R, R)]],
                                  pos.at[b], sem_p.at[b]).wait()

        def fire_out(c, b):
            pltpu.async_copy(outb.at[b], out_hbm.at[pl.ds(base + c * R, R)],
                             sem_o.at[b])

        def wait_out(c, b):
            pltpu.make_async_copy(outb.at[b],
                                  out_hbm.at[pl.ds(base + c * R, R)],
                                  sem_o.at[b]).wait()

        def fma(b):
            def row_body(r, carry):
                for cv in range(D // L):
                    s_ = pl.ds(cv * L, L)
                    outb[b, r, s_] = rows[b, r, s_] * scale + pos[b, r, s_]
                return carry
            lax.fori_loop(0, R, row_body, 0)

        for b in range(NBUF):
            fire_gather(b, b)

        for b in range(NBUF):
            wait_gather(b, b)
            fma(b)
            fire_out(b, b)
            fire_gather(b + NBUF, b)

        def group_body(g, carry):
            for b in range(NBUF):
                c = g * NBUF + b
                wait_gather(c, b)
                wait_out(c - NBUF, b)
                fma(b)
                fire_out(c, b)
                fire_gather(c + NBUF, b)
            return carry

        lax.fori_loop(1, n_groups - 1, group_body, 0)

        for b in range(NBUF):
            c = (n_groups - 1) * NBUF + b
            wait_gather(c, b)
            wait_out(c - NBUF, b)
            fma(b)
            fire_out(c, b)
        for b in range(NBUF):
            wait_out((n_groups - 1) * NBUF + b, b)

    return k


def kernel(input, embeddings, step=0):
    bs, sl = input.shape
    d = embeddings.shape[1]
    BT = bs * sl
    idx_flat = input.reshape(BT).astype(jnp.int32)
    pe = _pos_embedding(MAX_SEQ_LEN, d)
    pe_sl = lax.dynamic_slice_in_dim(pe, step, sl, axis=0)
    pe_aug = jnp.concatenate([pe_sl, jnp.zeros((8, d), jnp.float32)], axis=0)
    out_flat = _make_sc_kernel(BT, d, sl)(idx_flat, embeddings, pe_aug)
    return out_flat.reshape(bs, sl, d)
